# Initial kernel scaffold; baseline (speedup 1.0000x reference)
import jax
import jax.numpy as jnp
from jax.experimental import pallas as pl


def kernel(x, edge_index, edge_attr, W_self, W_neigh, b):
    raise NotImplementedError("write your pallas kernel here")


# speedup vs baseline: 5.2352x; 5.2352x over previous
import functools

import jax
import jax.numpy as jnp
from jax import lax
from jax.experimental import pallas as pl
from jax.experimental.pallas import tpu as pltpu
from jax.experimental.pallas import tpu_sc as plsc

N_NODES = 10000
N_EDGES = 320000
D_FEAT = 128
N_REL = 4

N_CHUNK = 4
CW = D_FEAT // N_CHUNK
NC = 2
NS = 16
KB = 128
NB = 158
EPW = NB * KB
E_PAD = NS * EPW
ACC_ROWS = 40448
TRASH_ROW = N_REL * N_NODES
ZROWS = ACC_ROWS // NS

BN = 400


def _sc_body(xt_hbm, src_hbm, scat_hbm, zeros_hbm, acc_hbm,
             sidx, didx, rows, accs, sem):
    c = lax.axis_index("c")
    s = lax.axis_index("s")
    pltpu.sync_copy(scat_hbm.at[s], didx)
    for j in range(2):
        ch = c * 2 + j
        pltpu.sync_copy(zeros_hbm, accs.at[pl.ds(s * ZROWS, ZROWS)])
        pltpu.sync_copy(src_hbm.at[ch, s], sidx)
        plsc.subcore_barrier()

        def body(b, carry):
            pltpu.async_copy(xt_hbm.at[sidx.at[b]], rows, sem).wait()
            pltpu.sync_copy(rows, accs.at[didx.at[b]], add=True)
            return carry

        lax.fori_loop(0, NB, body, 0)
        plsc.subcore_barrier()
        pltpu.sync_copy(accs.at[pl.ds(s * ZROWS, ZROWS)],
                        acc_hbm.at[ch, pl.ds(s * ZROWS, ZROWS)])


@functools.cache
def _sc_scatter():
    return pl.kernel(
        _sc_body,
        out_type=jax.ShapeDtypeStruct((N_CHUNK, ACC_ROWS, CW), jnp.float32),
        mesh=plsc.VectorSubcoreMesh(core_axis_name="c", subcore_axis_name="s"),
        scratch_types=[
            pltpu.VMEM((NB, KB), jnp.int32),
            pltpu.VMEM((NB, KB), jnp.int32),
            pltpu.VMEM((KB, CW), jnp.float32),
            pltpu.VMEM_SHARED((ACC_ROWS, CW), jnp.float32),
            pltpu.SemaphoreType.DMA,
        ],
        compiler_params=pltpu.CompilerParams(use_tc_tiling_on_sc=False),
    )


def _tc_body(x_ref, acc_ref, ws_ref, wn_ref, b_ref, out_ref, conv_ref):
    r = pl.program_id(1)
    c = pl.program_id(2)

    @pl.when((r == 0) & (c == 0))
    def _():
        out_ref[...] = jnp.zeros_like(out_ref)

    @pl.when(c == 0)
    def _():
        conv_ref[...] = (
            jnp.dot(x_ref[...], ws_ref[0], preferred_element_type=jnp.float32)
            + b_ref[0]
        )

    conv_ref[...] += jnp.dot(acc_ref[0], wn_ref[0, 0],
                             preferred_element_type=jnp.float32)

    @pl.when(c == N_CHUNK - 1)
    def _():
        out_ref[...] += jnp.tanh(conv_ref[...])


_tc_dense = pl.pallas_call(
    _tc_body,
    grid=(N_NODES // BN, N_REL, N_CHUNK),
    in_specs=[
        pl.BlockSpec((BN, D_FEAT), lambda nb, r, c: (nb, 0)),
        pl.BlockSpec((1, BN, CW),
                     lambda nb, r, c: (c, r * (N_NODES // BN) + nb, 0)),
        pl.BlockSpec((1, D_FEAT, D_FEAT), lambda nb, r, c: (r, 0, 0)),
        pl.BlockSpec((1, 1, CW, D_FEAT), lambda nb, r, c: (r, c, 0, 0)),
        pl.BlockSpec((1, 1, D_FEAT), lambda nb, r, c: (r, 0, 0)),
    ],
    out_specs=pl.BlockSpec((BN, D_FEAT), lambda nb, r, c: (nb, 0)),
    out_shape=jax.ShapeDtypeStruct((N_NODES, D_FEAT), jnp.float32),
    scratch_shapes=[pltpu.VMEM((BN, D_FEAT), jnp.float32)],
    compiler_params=pltpu.CompilerParams(
        dimension_semantics=("arbitrary", "arbitrary", "arbitrary")),
)


def kernel(x, edge_index, edge_attr, W_self, W_neigh, b):
    src = edge_index[0]
    dst = edge_index[1]
    xt = x.reshape(N_NODES, N_CHUNK, CW).transpose(1, 0, 2)
    xt = xt.reshape(N_CHUNK * N_NODES, CW)
    pad = E_PAD - N_EDGES
    srcp = jnp.concatenate([src, jnp.zeros((pad,), jnp.int32)])
    src4 = srcp[None, :] + (jnp.arange(N_CHUNK, dtype=jnp.int32)
                            * N_NODES)[:, None]
    src4 = src4.reshape(N_CHUNK, NS, NB, KB)
    scat = jnp.concatenate(
        [edge_attr * N_NODES + dst,
         jnp.full((pad,), TRASH_ROW, jnp.int32)]).reshape(NS, NB, KB)
    zeros_z = jnp.zeros((ZROWS, CW), jnp.float32)

    acc = _sc_scatter()(xt, src4, scat, zeros_z)

    wn4 = W_neigh.reshape(N_REL, N_CHUNK, CW, D_FEAT)
    return _tc_dense(x, acc, W_self, wn4, b.reshape(N_REL, 1, D_FEAT))


# speedup vs baseline: 6.4833x; 1.2384x over previous
import functools

import jax
import jax.numpy as jnp
from jax import lax
from jax.experimental import pallas as pl
from jax.experimental.pallas import tpu as pltpu
from jax.experimental.pallas import tpu_sc as plsc

N_NODES = 10000
N_EDGES = 320000
D_FEAT = 128
N_REL = 4

N_CHUNK = 4
CW = D_FEAT // N_CHUNK
NC = 2
NS = 16
KB = 128
NB = 158
EPW = NB * KB
E_PAD = NS * EPW
ACC_ROWS = 40448
TRASH_ROW = N_REL * N_NODES
ZROWS = ACC_ROWS // NS

BN = 400


def _sc_body(xt_hbm, src_hbm, scat_hbm, zeros_hbm, acc_hbm,
             sidx, didx, rows0, rows1, accs, sem0, sem1):
    c = lax.axis_index("c")
    s = lax.axis_index("s")
    pltpu.sync_copy(scat_hbm.at[s], didx)
    for j in range(2):
        ch = c * 2 + j
        pltpu.sync_copy(zeros_hbm, accs.at[pl.ds(s * ZROWS, ZROWS)])
        pltpu.sync_copy(src_hbm.at[ch, s], sidx)
        plsc.subcore_barrier()

        pltpu.async_copy(xt_hbm.at[sidx.at[0]], rows0, sem0)

        def body(k, carry):
            b0 = 2 * k
            pltpu.async_copy(xt_hbm.at[sidx.at[b0 + 1]], rows1, sem1)
            pltpu.make_async_copy(xt_hbm.at[sidx.at[b0]], rows0, sem0).wait()
            pltpu.sync_copy(rows0, accs.at[didx.at[b0]], add=True)

            @pl.when(b0 + 2 < NB)
            def _():
                pltpu.async_copy(xt_hbm.at[sidx.at[b0 + 2]], rows0, sem0)

            pltpu.make_async_copy(xt_hbm.at[sidx.at[b0 + 1]], rows1,
                                  sem1).wait()
            pltpu.sync_copy(rows1, accs.at[didx.at[b0 + 1]], add=True)
            return carry

        lax.fori_loop(0, NB // 2, body, 0)
        plsc.subcore_barrier()
        pltpu.sync_copy(accs.at[pl.ds(s * ZROWS, ZROWS)],
                        acc_hbm.at[ch, pl.ds(s * ZROWS, ZROWS)])


@functools.cache
def _sc_scatter():
    return pl.kernel(
        _sc_body,
        out_type=jax.ShapeDtypeStruct((N_CHUNK, ACC_ROWS, CW), jnp.float32),
        mesh=plsc.VectorSubcoreMesh(core_axis_name="c", subcore_axis_name="s"),
        scratch_types=[
            pltpu.VMEM((NB, KB), jnp.int32),
            pltpu.VMEM((NB, KB), jnp.int32),
            pltpu.VMEM((KB, CW), jnp.float32),
            pltpu.VMEM((KB, CW), jnp.float32),
            pltpu.VMEM_SHARED((ACC_ROWS, CW), jnp.float32),
            pltpu.SemaphoreType.DMA,
            pltpu.SemaphoreType.DMA,
        ],
        compiler_params=pltpu.CompilerParams(use_tc_tiling_on_sc=False),
    )


def _tc_body(x_ref, acc_ref, ws_ref, wn_ref, b_ref, out_ref, conv_ref):
    r = pl.program_id(1)
    c = pl.program_id(2)

    @pl.when((r == 0) & (c == 0))
    def _():
        out_ref[...] = jnp.zeros_like(out_ref)

    @pl.when(c == 0)
    def _():
        conv_ref[...] = (
            jnp.dot(x_ref[...], ws_ref[0], preferred_element_type=jnp.float32)
            + b_ref[0]
        )

    conv_ref[...] += jnp.dot(acc_ref[0], wn_ref[0, 0],
                             preferred_element_type=jnp.float32)

    @pl.when(c == N_CHUNK - 1)
    def _():
        out_ref[...] += jnp.tanh(conv_ref[...])


_tc_dense = pl.pallas_call(
    _tc_body,
    grid=(N_NODES // BN, N_REL, N_CHUNK),
    in_specs=[
        pl.BlockSpec((BN, D_FEAT), lambda nb, r, c: (nb, 0)),
        pl.BlockSpec((1, BN, CW),
                     lambda nb, r, c: (c, r * (N_NODES // BN) + nb, 0)),
        pl.BlockSpec((1, D_FEAT, D_FEAT), lambda nb, r, c: (r, 0, 0)),
        pl.BlockSpec((1, 1, CW, D_FEAT), lambda nb, r, c: (r, c, 0, 0)),
        pl.BlockSpec((1, 1, D_FEAT), lambda nb, r, c: (r, 0, 0)),
    ],
    out_specs=pl.BlockSpec((BN, D_FEAT), lambda nb, r, c: (nb, 0)),
    out_shape=jax.ShapeDtypeStruct((N_NODES, D_FEAT), jnp.float32),
    scratch_shapes=[pltpu.VMEM((BN, D_FEAT), jnp.float32)],
    compiler_params=pltpu.CompilerParams(
        dimension_semantics=("arbitrary", "arbitrary", "arbitrary")),
)


def kernel(x, edge_index, edge_attr, W_self, W_neigh, b):
    src = edge_index[0]
    dst = edge_index[1]
    xt = x.reshape(N_NODES, N_CHUNK, CW).transpose(1, 0, 2)
    xt = xt.reshape(N_CHUNK * N_NODES, CW)
    pad = E_PAD - N_EDGES
    srcp = jnp.concatenate([src, jnp.zeros((pad,), jnp.int32)])
    src4 = srcp[None, :] + (jnp.arange(N_CHUNK, dtype=jnp.int32)
                            * N_NODES)[:, None]
    src4 = src4.reshape(N_CHUNK, NS, NB, KB)
    scat = jnp.concatenate(
        [edge_attr * N_NODES + dst,
         jnp.full((pad,), TRASH_ROW, jnp.int32)]).reshape(NS, NB, KB)
    zeros_z = jnp.zeros((ZROWS, CW), jnp.float32)

    acc = _sc_scatter()(xt, src4, scat, zeros_z)

    wn4 = W_neigh.reshape(N_REL, N_CHUNK, CW, D_FEAT)
    return _tc_dense(x, acc, W_self, wn4, b.reshape(N_REL, 1, D_FEAT))


# speedup vs baseline: 10.2080x; 1.5745x over previous
import functools

import jax
import jax.numpy as jnp
from jax import lax
from jax.experimental import pallas as pl
from jax.experimental.pallas import tpu as pltpu
from jax.experimental.pallas import tpu_sc as plsc

N_NODES = 10000
N_EDGES = 320000
D_FEAT = 128
N_REL = 4

N_CHUNK = 4
CW = D_FEAT // N_CHUNK
NC = 2
NS = 16
KB = 128
NB = 158
EPW = NB * KB
E_PAD = NS * EPW
ACC_ROWS = 40448
TRASH_ROW = N_REL * N_NODES
ZROWS = ACC_ROWS // NS

BN = 400


def _sc_body(xt_hbm, src_hbm, scat_hbm, zeros_hbm, acc_hbm,
             sidx, didx, rows0, rows1, accs, sem0, sem1):
    c = lax.axis_index("c")
    s = lax.axis_index("s")
    pltpu.sync_copy(scat_hbm.at[s], didx)
    for j in range(2):
        ch = c * 2 + j
        pltpu.sync_copy(zeros_hbm, accs.at[pl.ds(s * ZROWS, ZROWS)])
        pltpu.sync_copy(src_hbm.at[ch, s], sidx)
        plsc.subcore_barrier()

        pltpu.async_copy(xt_hbm.at[sidx.at[0]], rows0, sem0)

        def body(k, carry):
            b0 = 2 * k
            pltpu.async_copy(xt_hbm.at[sidx.at[b0 + 1]], rows1, sem1)
            pltpu.make_async_copy(xt_hbm.at[sidx.at[b0]], rows0, sem0).wait()
            pltpu.sync_copy(rows0, accs.at[didx.at[b0]], add=True)

            @pl.when(b0 + 2 < NB)
            def _():
                pltpu.async_copy(xt_hbm.at[sidx.at[b0 + 2]], rows0, sem0)

            pltpu.make_async_copy(xt_hbm.at[sidx.at[b0 + 1]], rows1,
                                  sem1).wait()
            pltpu.sync_copy(rows1, accs.at[didx.at[b0 + 1]], add=True)
            return carry

        lax.fori_loop(0, NB // 2, body, 0)
        plsc.subcore_barrier()
        pltpu.sync_copy(accs.at[pl.ds(s * ZROWS, ZROWS)],
                        acc_hbm.at[pl.ds(s * ZROWS, ZROWS),
                                   pl.ds(ch * CW, CW)])


@functools.cache
def _sc_scatter():
    return pl.kernel(
        _sc_body,
        out_type=jax.ShapeDtypeStruct((ACC_ROWS, D_FEAT), jnp.float32),
        mesh=plsc.VectorSubcoreMesh(core_axis_name="c", subcore_axis_name="s"),
        scratch_types=[
            pltpu.VMEM((NB, KB), jnp.int32),
            pltpu.VMEM((NB, KB), jnp.int32),
            pltpu.VMEM((KB, CW), jnp.float32),
            pltpu.VMEM((KB, CW), jnp.float32),
            pltpu.VMEM_SHARED((ACC_ROWS, CW), jnp.float32),
            pltpu.SemaphoreType.DMA,
            pltpu.SemaphoreType.DMA,
        ],
        compiler_params=pltpu.CompilerParams(use_tc_tiling_on_sc=False),
    )


def _tc_body(x_ref, acc_ref, ws_ref, wn_ref, b_ref, out_ref):
    r = pl.program_id(1)

    @pl.when(r == 0)
    def _():
        out_ref[...] = jnp.zeros_like(out_ref)

    conv = (jnp.dot(x_ref[...], ws_ref[0], preferred_element_type=jnp.float32)
            + jnp.dot(acc_ref[...], wn_ref[0],
                      preferred_element_type=jnp.float32)
            + b_ref[0])
    out_ref[...] += jnp.tanh(conv)


_tc_dense = pl.pallas_call(
    _tc_body,
    grid=(N_NODES // BN, N_REL),
    in_specs=[
        pl.BlockSpec((BN, D_FEAT), lambda nb, r: (nb, 0)),
        pl.BlockSpec((BN, D_FEAT),
                     lambda nb, r: (r * (N_NODES // BN) + nb, 0)),
        pl.BlockSpec((1, D_FEAT, D_FEAT), lambda nb, r: (r, 0, 0)),
        pl.BlockSpec((1, D_FEAT, D_FEAT), lambda nb, r: (r, 0, 0)),
        pl.BlockSpec((1, 1, D_FEAT), lambda nb, r: (r, 0, 0)),
    ],
    out_specs=pl.BlockSpec((BN, D_FEAT), lambda nb, r: (nb, 0)),
    out_shape=jax.ShapeDtypeStruct((N_NODES, D_FEAT), jnp.float32),
    compiler_params=pltpu.CompilerParams(
        dimension_semantics=("arbitrary", "arbitrary")),
)


def kernel(x, edge_index, edge_attr, W_self, W_neigh, b):
    src = edge_index[0]
    dst = edge_index[1]
    xt = x.reshape(N_NODES, N_CHUNK, CW).transpose(1, 0, 2)
    xt = xt.reshape(N_CHUNK * N_NODES, CW)
    pad = E_PAD - N_EDGES
    srcp = jnp.concatenate([src, jnp.zeros((pad,), jnp.int32)])
    src4 = srcp[None, :] + (jnp.arange(N_CHUNK, dtype=jnp.int32)
                            * N_NODES)[:, None]
    src4 = src4.reshape(N_CHUNK, NS, NB, KB)
    scat = jnp.concatenate(
        [edge_attr * N_NODES + dst,
         jnp.full((pad,), TRASH_ROW, jnp.int32)]).reshape(NS, NB, KB)
    zeros_z = jnp.zeros((ZROWS, CW), jnp.float32)

    acc = _sc_scatter()(xt, src4, scat, zeros_z)

    return _tc_dense(x, acc, W_self, W_neigh, b.reshape(N_REL, 1, D_FEAT))
